# baseline (speedup 1.0000x reference)
import functools

import jax
import jax.numpy as jnp
from jax import lax
from jax.experimental import pallas as pl
from jax.experimental.pallas import tpu as pltpu
from jax.experimental.pallas import tpu_sc as plsc

E = 32768
LBD = 0.1
L = 16

LN2 = 0.6931471805599453
SQRT2 = 1.4142135623730951
CM = [2.8611005965088914e-08, 0.9999998571599664, -0.5000094528500123,
      0.3333570973747606, -0.2495212848772336, 0.19882802594237414,
      -0.17405911832439713, 0.16365879955638582, -0.09842353538796306]
CP = [5.239402951757033e-09, 0.9999989105817847, -0.4999622445170655,
      0.33281842539712797, -0.24635660615435243, 0.18468848457174256,
      -0.12526661430202796, 0.06651247927615206, -0.023038279921030243,
      0.0037526242132415377]

_GDN = lax.GatherDimensionNumbers(
    offset_dims=(), collapsed_slice_dims=(0,), start_index_map=(0,))


def _dgather(v, idx):
    return lax.gather(v, idx[:, None], _GDN, (1,),
                      mode=lax.GatherScatterMode.PROMISE_IN_BOUNDS)


def _lane_total(v, lane):
    for sh in (8, 4, 2, 1):
        v = v + _dgather(v, jnp.bitwise_xor(lane, sh))
    return v


def _horner(c, x):
    r = jnp.full_like(x, c[-1])
    for v in c[-2::-1]:
        r = r * x + jnp.float32(v)
    return r


def _flog(x):
    xi = lax.bitcast_convert_type(x, jnp.int32)
    ex = jnp.bitwise_and(lax.shift_right_logical(xi, 23), 0xFF) - 127
    mb = jnp.bitwise_or(jnp.bitwise_and(xi, 0x007FFFFF), 0x3F800000)
    m = lax.bitcast_convert_type(mb, jnp.float32)
    big = m > SQRT2
    m = jnp.where(big, m * 0.5, m)
    ex = jnp.where(big, ex + 1, ex).astype(jnp.float32)
    return _horner(CM, m - 1.0) + ex * LN2


def _make_sc_kernel():
    info = plsc.get_sparse_core_info()
    nc, ns = info.num_cores, info.num_subcores
    nw = nc * ns
    chunk = E // nw
    nvec = chunk // L
    mesh = plsc.VectorSubcoreMesh(core_axis_name="c", subcore_axis_name="s")

    @functools.partial(
        pl.kernel,
        mesh=mesh,
        out_type=jax.ShapeDtypeStruct((nw, L), jnp.float32),
        scratch_types=[
            pltpu.VMEM((chunk,), jnp.float32),
            pltpu.VMEM((chunk,), jnp.float32),
            pltpu.VMEM((chunk,), jnp.float32),
            pltpu.VMEM((chunk,), jnp.int32),
            pltpu.VMEM((L,), jnp.float32),
            pltpu.SemaphoreType.DMA,
            pltpu.SemaphoreType.DMA,
            pltpu.SemaphoreType.DMA,
            pltpu.SemaphoreType.DMA,
        ],
    )
    def sc_kernel(x0_hbm, l0_hbm, l1_hbm, t_hbm, out_hbm, x0_v, l0_v, l1_v,
                  t_v, part_v, sem0, sem1, sem2, sem3):
        wid = lax.axis_index("s") * nc + lax.axis_index("c")
        base = wid * chunk
        cp0 = pltpu.async_copy(x0_hbm.at[pl.ds(base, chunk)], x0_v, sem0)
        cp1 = pltpu.async_copy(l0_hbm.at[pl.ds(base, chunk)], l0_v, sem1)
        cp2 = pltpu.async_copy(l1_hbm.at[pl.ds(base, chunk)], l1_v, sem2)
        cp3 = pltpu.async_copy(t_hbm.at[pl.ds(base, chunk)], t_v, sem3)
        cp0.wait()
        cp1.wait()
        cp2.wait()
        cp3.wait()

        def step(i, carry):
            kl_a, ax_a, ac_a = carry
            x0 = x0_v[pl.ds(i * L, L)]
            l0 = l0_v[pl.ds(i * L, L)]
            l1 = l1_v[pl.ds(i * L, L)]
            tf = t_v[pl.ds(i * L, L)].astype(jnp.float32)

            kt = 1.0 - tf * (1.0 / 64.0)
            ktm1 = kt + (1.0 / 64.0)
            bt = kt / ktm1

            s1 = 1.0 / (1.0 + jnp.exp(l0 - l1))
            s0 = 1.0 - s1

            w = (x0 * kt) * bt
            q0 = 1.0 - w
            pr0 = (s0 * ktm1 + (1.0 - ktm1)) * q0
            pr1 = (s1 * ktm1) * w
            rs = 1.0 / (pr0 + pr1 + 1e-6)
            ftr = tf == 1.0
            tm10 = jnp.where(ftr, s0, pr0 * rs)
            tm11 = jnp.where(ftr, s1, pr1 * rs)
            u0 = ((1.0 - x0) * ktm1 + (1.0 - ktm1)) * q0
            u1 = (x0 * ktm1) * w
            us = 1.0 / (u0 + u1 + 1e-6)
            g0 = jnp.clip(u0 * us, 0.0, 1.0)
            g1 = jnp.clip(u1 * us, 0.0, 1.0)
            p0 = jnp.clip(tm10, 1e-6, 1.0 - 1e-6)
            p1 = jnp.clip(tm11, 1e-6, 1.0 - 1e-6)
            lp0 = _flog(p0)
            lp1 = _flog(p1)
            aux = -((g0 + 1.0 - g1) * lp0 + (1.0 - g0 + g1) * lp1)

            kl0 = (jnp.maximum(l0, 0.0) - l0 * (1.0 - x0)
                   + _horner(CP, jnp.exp(-jnp.abs(l0))))
            kl1 = (jnp.maximum(l1, 0.0) - l1 * x0
                   + _horner(CP, jnp.exp(-jnp.abs(l1))))

            af = jnp.where(l1 > l0, 1.0, 0.0)
            accv = jnp.where(af == x0, 1.0, 0.0)
            return kl_a + (kl0 + kl1), ax_a + aux, ac_a + accv

        zero = jnp.zeros((L,), jnp.float32)
        kl_s, ax_s, ac_s = plsc.parallel_loop(
            0, nvec, unroll=4, carry=(zero, zero, zero))(step)
        lane = lax.iota(jnp.int32, L)
        kl_t = _lane_total(kl_s, lane)
        ax_t = _lane_total(ax_s, lane)
        ac_t = _lane_total(ac_s, lane)
        comb = (jnp.where(lane == 0, kl_t, zero)
                + jnp.where(lane == 1, ax_t, zero)
                + jnp.where(lane == 2, ac_t, zero))
        part_v[...] = comb
        pltpu.sync_copy(part_v, out_hbm.at[wid])

    return sc_kernel


def kernel(full_edge_0, full_edge_0_hat_logits, t_edge):
    sc = _make_sc_kernel()
    l0 = full_edge_0_hat_logits[:, 0]
    l1 = full_edge_0_hat_logits[:, 1]
    parts = sc(full_edge_0, l0, l1, t_edge)
    sums = parts.sum(axis=0)
    kl_loss = sums[0] / (2.0 * E)
    aux_loss = sums[1] / (2.0 * E)
    acc = sums[2] / E
    total = LBD * aux_loss + kl_loss
    return (total, kl_loss, acc, aux_loss)


# speedup vs baseline: 1.0011x; 1.0011x over previous
import functools

import jax
import jax.numpy as jnp
from jax import lax
from jax.experimental import pallas as pl
from jax.experimental.pallas import tpu as pltpu
from jax.experimental.pallas import tpu_sc as plsc

E = 32768
LBD = 0.1
L = 16

LN2 = 0.6931471805599453
SQRT2 = 1.4142135623730951
CM = [2.8611005965088914e-08, 0.9999998571599664, -0.5000094528500123,
      0.3333570973747606, -0.2495212848772336, 0.19882802594237414,
      -0.17405911832439713, 0.16365879955638582, -0.09842353538796306]
CP = [5.239402951757033e-09, 0.9999989105817847, -0.4999622445170655,
      0.33281842539712797, -0.24635660615435243, 0.18468848457174256,
      -0.12526661430202796, 0.06651247927615206, -0.023038279921030243,
      0.0037526242132415377]

_GDN = lax.GatherDimensionNumbers(
    offset_dims=(), collapsed_slice_dims=(0,), start_index_map=(0,))


def _dgather(v, idx):
    return lax.gather(v, idx[:, None], _GDN, (1,),
                      mode=lax.GatherScatterMode.PROMISE_IN_BOUNDS)


def _lane_total(v, lane):
    for sh in (8, 4, 2, 1):
        v = v + _dgather(v, jnp.bitwise_xor(lane, sh))
    return v


def _horner(c, x):
    r = jnp.full_like(x, c[-1])
    for v in c[-2::-1]:
        r = r * x + jnp.float32(v)
    return r


def _flog(x):
    xi = lax.bitcast_convert_type(x, jnp.int32)
    ex = jnp.bitwise_and(lax.shift_right_logical(xi, 23), 0xFF) - 127
    mb = jnp.bitwise_or(jnp.bitwise_and(xi, 0x007FFFFF), 0x3F800000)
    m = lax.bitcast_convert_type(mb, jnp.float32)
    big = m > SQRT2
    m = jnp.where(big, m * 0.5, m)
    ex = jnp.where(big, ex + 1, ex).astype(jnp.float32)
    return _horner(CM, m - 1.0) + ex * LN2


def _make_sc_kernel():
    info = plsc.get_sparse_core_info()
    nc, ns = info.num_cores, info.num_subcores
    nw = nc * ns
    chunk = E // nw
    nvec = chunk // L
    mesh = plsc.VectorSubcoreMesh(core_axis_name="c", subcore_axis_name="s")

    @functools.partial(
        pl.kernel,
        mesh=mesh,
        out_type=jax.ShapeDtypeStruct((nw, L), jnp.float32),
        scratch_types=[
            pltpu.VMEM((chunk,), jnp.float32),
            pltpu.VMEM((chunk,), jnp.float32),
            pltpu.VMEM((chunk,), jnp.float32),
            pltpu.VMEM((chunk,), jnp.int32),
            pltpu.VMEM((L,), jnp.float32),
            pltpu.SemaphoreType.DMA,
            pltpu.SemaphoreType.DMA,
            pltpu.SemaphoreType.DMA,
            pltpu.SemaphoreType.DMA,
        ],
    )
    def sc_kernel(x0_hbm, l0_hbm, l1_hbm, t_hbm, out_hbm, x0_v, l0_v, l1_v,
                  t_v, part_v, sem0, sem1, sem2, sem3):
        wid = lax.axis_index("s") * nc + lax.axis_index("c")
        base = wid * chunk
        cp0 = pltpu.async_copy(x0_hbm.at[pl.ds(base, chunk)], x0_v, sem0)
        cp1 = pltpu.async_copy(l0_hbm.at[pl.ds(base, chunk)], l0_v, sem1)
        cp2 = pltpu.async_copy(l1_hbm.at[pl.ds(base, chunk)], l1_v, sem2)
        cp3 = pltpu.async_copy(t_hbm.at[pl.ds(base, chunk)], t_v, sem3)
        cp0.wait()
        cp1.wait()
        cp2.wait()
        cp3.wait()

        def step(i, carry):
            kl_a, ax_a, ac_a = carry
            x0 = x0_v[pl.ds(i * L, L)]
            l0 = l0_v[pl.ds(i * L, L)]
            l1 = l1_v[pl.ds(i * L, L)]
            tf = t_v[pl.ds(i * L, L)].astype(jnp.float32)

            kt = 1.0 - tf * (1.0 / 64.0)
            ktm1 = kt + (1.0 / 64.0)
            bt = kt / ktm1

            s1 = 1.0 / (1.0 + jnp.exp(l0 - l1))
            s0 = 1.0 - s1

            w = (x0 * kt) * bt
            q0 = 1.0 - w
            pr0 = (s0 * ktm1 + (1.0 - ktm1)) * q0
            pr1 = (s1 * ktm1) * w
            rs = 1.0 / (pr0 + pr1 + 1e-6)
            ftr = tf == 1.0
            tm10 = jnp.where(ftr, s0, pr0 * rs)
            tm11 = jnp.where(ftr, s1, pr1 * rs)
            u0 = ((1.0 - x0) * ktm1 + (1.0 - ktm1)) * q0
            u1 = (x0 * ktm1) * w
            us = 1.0 / (u0 + u1 + 1e-6)
            g0 = jnp.clip(u0 * us, 0.0, 1.0)
            g1 = jnp.clip(u1 * us, 0.0, 1.0)
            p0 = jnp.clip(tm10, 1e-6, 1.0 - 1e-6)
            p1 = jnp.clip(tm11, 1e-6, 1.0 - 1e-6)
            lp0 = _flog(p0)
            lp1 = _flog(p1)
            aux = -((g0 + 1.0 - g1) * lp0 + (1.0 - g0 + g1) * lp1)

            kl0 = (jnp.maximum(l0, 0.0) - l0 * (1.0 - x0)
                   + _horner(CP, jnp.exp(-jnp.abs(l0))))
            kl1 = (jnp.maximum(l1, 0.0) - l1 * x0
                   + _horner(CP, jnp.exp(-jnp.abs(l1))))

            af = jnp.where(l1 > l0, 1.0, 0.0)
            accv = jnp.where(af == x0, 1.0, 0.0)
            return kl_a + (kl0 + kl1), ax_a + aux, ac_a + accv

        zero = jnp.zeros((L,), jnp.float32)
        kl_s, ax_s, ac_s = plsc.parallel_loop(
            0, nvec, unroll=8, carry=(zero, zero, zero))(step)
        lane = lax.iota(jnp.int32, L)
        kl_t = _lane_total(kl_s, lane)
        ax_t = _lane_total(ax_s, lane)
        ac_t = _lane_total(ac_s, lane)
        comb = (jnp.where(lane == 0, kl_t, zero)
                + jnp.where(lane == 1, ax_t, zero)
                + jnp.where(lane == 2, ac_t, zero))
        part_v[...] = comb
        pltpu.sync_copy(part_v, out_hbm.at[wid])

    return sc_kernel


def kernel(full_edge_0, full_edge_0_hat_logits, t_edge):
    sc = _make_sc_kernel()
    l0 = full_edge_0_hat_logits[:, 0]
    l1 = full_edge_0_hat_logits[:, 1]
    parts = sc(full_edge_0, l0, l1, t_edge)
    sums = parts.sum(axis=0)
    kl_loss = sums[0] / (2.0 * E)
    aux_loss = sums[1] / (2.0 * E)
    acc = sums[2] / E
    total = LBD * aux_loss + kl_loss
    return (total, kl_loss, acc, aux_loss)
